# speedup vs baseline: 2.3289x; 2.3289x over previous
import functools

import jax
import jax.numpy as jnp
from jax import lax
from jax.experimental import pallas as pl
from jax.experimental.pallas import tpu as pltpu
from jax.experimental.pallas import tpu_sc as plsc

_D = 128
_B = 16384
_NC = 2
_NS = 16
_NW = _NC * _NS
_BPW = _B // _NW
_CHUNK = 128
_NCH = _BPW // _CHUNK

_mesh = plsc.VectorSubcoreMesh(core_axis_name="c", subcore_axis_name="s")


@functools.partial(
    pl.kernel,
    mesh=_mesh,
    out_type=jax.ShapeDtypeStruct((_B // _CHUNK, _CHUNK, _D), jnp.float32),
    scratch_types=[
        pltpu.VMEM((_NCH, _CHUNK), jnp.int32),
        pltpu.VMEM((_NCH, _CHUNK, _D), jnp.float32),
        pltpu.SemaphoreType.DMA,
        pltpu.SemaphoreType.DMA,
    ],
)
def _sc_gather(labels_hbm, table_hbm, out_hbm, idx_v, rows_v, gsem, osem):
    wid = lax.axis_index("s") * _NC + lax.axis_index("c")
    pltpu.sync_copy(labels_hbm.at[wid], idx_v)
    gathers = [
        pltpu.async_copy(table_hbm.at[idx_v.at[j]], rows_v.at[j], gsem)
        for j in range(_NCH)
    ]
    outs = []
    for j in range(_NCH):
        gathers[j].wait()
        outs.append(
            pltpu.async_copy(rows_v.at[j], out_hbm.at[wid * _NCH + j], osem)
        )
    for o in outs:
        o.wait()


def kernel(labels, train, dtype, table):
    del train
    labels3 = labels.astype(jnp.int32).reshape(_NW, _NCH, _CHUNK)
    out = _sc_gather(labels3, table)
    return out.reshape(_B, _D).astype(dtype.dtype)
